# baseline (speedup 1.0000x reference)
import jax
import jax.numpy as jnp
from jax.experimental import pallas as pl
from jax.experimental.pallas import tpu as pltpu

_D = 1024
_H = 16
_KS = 3
_K = 64
_L = 2048
_LOUT = _L - _KS + 1
_BISECT_ITERS = 32
_TD = 128
_J = _D // _TD


def _roll(x, shift):
    return pltpu.roll(x, shift % x.shape[1], 1)


def _compute_w(src, W, p_ref):
    Y = jax.lax.dot_general(W, src, (((1,), (0,)), ((), ())),
                            preferred_element_type=jnp.float32)
    b_att = p_ref[:, 0:1]
    gamma = p_ref[:, 1:2]
    beta = p_ref[:, 2:3]
    wc = p_ref[:, 3:4]
    xh = Y[0:_H] + _roll(Y[_H:2 * _H], -1) + _roll(Y[2 * _H:3 * _H], -2) + b_att
    lane = jax.lax.broadcasted_iota(jnp.int32, (_H, _L), 1)
    valid = lane < _LOUT
    cnt = jnp.float32(_LOUT)
    xv = jnp.where(valid, xh, 0.0)
    mean = jnp.sum(xv, axis=1, keepdims=True) / cnt
    dx = jnp.where(valid, xh - mean, 0.0)
    var = jnp.sum(dx * dx, axis=1, keepdims=True) / cnt
    xn = (xh - mean) * jax.lax.rsqrt(var + 1e-5) * gamma + beta
    xm = jnp.where(valid, xn, jnp.float32(-jnp.inf))
    lo = jnp.min(jnp.where(valid, xn, jnp.float32(jnp.inf)), axis=1, keepdims=True)
    hi = jnp.max(xm, axis=1, keepdims=True)

    def bis(_, carry):
        lo, hi = carry
        mid = 0.5 * (lo + hi)
        c = jnp.sum(jnp.where(xm >= mid, 1.0, 0.0), axis=1, keepdims=True)
        ge = c >= jnp.float32(_K)
        return jnp.where(ge, mid, lo), jnp.where(ge, hi, mid)

    lo, hi = jax.lax.fori_loop(0, _BISECT_ITERS, bis, (lo, hi))
    mask = xm >= lo
    sm = jnp.where(mask, jax.nn.sigmoid(xn), 0.0)
    acc = sm + _roll(sm, 1) + _roll(sm, 2)
    return jnp.sum(acc * wc, axis=0, keepdims=True) * jnp.float32(1.0 / _KS)


def _body(src_ref, w_ref, p_ref, out_ref, wv_ref):
    j = pl.program_id(1)

    @pl.when(j == 0)
    def _():
        wv_ref[...] = _compute_w(src_ref[0], w_ref[...], p_ref)

    @pl.when(j > 0)
    def _():
        rows = src_ref[0, pl.ds((j - 1) * _TD, _TD), :]
        out_ref[0] = rows * wv_ref[...] + p_ref[0:1, 4:5]


def kernel(src_seqs, W_att, b_att, gamma, beta, W_comb, b_comb):
    B = src_seqs.shape[0]
    Wt = jnp.transpose(W_att, (2, 0, 1)).reshape(_KS * _H, _D)
    params = jnp.stack([b_att, gamma, beta, W_comb[0, :, 0],
                        jnp.full((_H,), b_comb[0], jnp.float32)], axis=1)
    params = jnp.pad(params, ((0, 0), (0, 3)))
    return pl.pallas_call(
        _body,
        grid=(B, 1 + _J),
        in_specs=[
            pl.BlockSpec((1, _D, _L), lambda b, j: (b, 0, 0)),
            pl.BlockSpec((_KS * _H, _D), lambda b, j: (0, 0)),
            pl.BlockSpec((_H, 8), lambda b, j: (0, 0)),
        ],
        out_specs=pl.BlockSpec((1, _TD, _L),
                               lambda b, j: (b, jnp.maximum(j - 1, 0), 0)),
        out_shape=jax.ShapeDtypeStruct(src_seqs.shape, jnp.float32),
        scratch_shapes=[pltpu.VMEM((1, _L), jnp.float32)],
    )(src_seqs, Wt, params)


# speedup vs baseline: 1.3275x; 1.3275x over previous
import jax
import jax.numpy as jnp
from jax.experimental import pallas as pl
from jax.experimental.pallas import tpu as pltpu

_D = 1024
_H = 16
_KS = 3
_K = 64
_L = 2048
_LOUT = _L - _KS + 1
_BISECT_ITERS = 32
_TD = 128
_J = _D // _TD


def _roll(x, shift):
    return pltpu.roll(x, shift % x.shape[1], 1)


def _compute_w(src, W, p_ref):
    Y = jax.lax.dot_general(W, src, (((1,), (0,)), ((), ())),
                            preferred_element_type=jnp.float32)
    b_att = p_ref[:, 0:1]
    gamma = p_ref[:, 1:2]
    beta = p_ref[:, 2:3]
    wc = p_ref[:, 3:4]
    xh = Y[0:_H] + _roll(Y[_H:2 * _H], -1) + _roll(Y[2 * _H:3 * _H], -2) + b_att
    lane = jax.lax.broadcasted_iota(jnp.int32, (_H, _L), 1)
    valid = lane < _LOUT
    cnt = jnp.float32(_LOUT)
    xv = jnp.where(valid, xh, 0.0)
    mean = jnp.sum(xv, axis=1, keepdims=True) / cnt
    dx = jnp.where(valid, xh - mean, 0.0)
    var = jnp.sum(dx * dx, axis=1, keepdims=True) / cnt
    xn = (xh - mean) * jax.lax.rsqrt(var + 1e-5) * gamma + beta
    xm = jnp.where(valid, xn, jnp.float32(-jnp.inf))
    lo = jnp.min(jnp.where(valid, xn, jnp.float32(jnp.inf)), axis=1, keepdims=True)
    hi = jnp.max(xm, axis=1, keepdims=True)

    def bis(_, carry):
        lo, hi = carry
        mid = 0.5 * (lo + hi)
        c = jnp.sum(jnp.where(xm >= mid, 1.0, 0.0), axis=1, keepdims=True)
        ge = c >= jnp.float32(_K)
        return jnp.where(ge, mid, lo), jnp.where(ge, hi, mid)

    lo, hi = jax.lax.fori_loop(0, _BISECT_ITERS, bis, (lo, hi))
    mask = xm >= lo
    sm = jnp.where(mask, jax.nn.sigmoid(xn), 0.0)
    acc = sm + _roll(sm, 1) + _roll(sm, 2)
    return jnp.sum(acc * wc, axis=0, keepdims=True) * jnp.float32(1.0 / _KS)


def _body(src_ref, w_ref, p_ref, out_ref):
    src = src_ref[0]
    wvec = _compute_w(src, w_ref[...], p_ref)
    out_ref[0] = src * wvec + p_ref[0:1, 4:5]


def kernel(src_seqs, W_att, b_att, gamma, beta, W_comb, b_comb):
    B = src_seqs.shape[0]
    Wt = jnp.transpose(W_att, (2, 0, 1)).reshape(_KS * _H, _D)
    params = jnp.stack([b_att, gamma, beta, W_comb[0, :, 0],
                        jnp.full((_H,), b_comb[0], jnp.float32)], axis=1)
    params = jnp.pad(params, ((0, 0), (0, 3)))
    return pl.pallas_call(
        _body,
        grid=(B,),
        in_specs=[
            pl.BlockSpec((1, _D, _L), lambda b: (b, 0, 0)),
            pl.BlockSpec((_KS * _H, _D), lambda b: (0, 0)),
            pl.BlockSpec((_H, 8), lambda b: (0, 0)),
        ],
        out_specs=pl.BlockSpec((1, _D, _L), lambda b: (b, 0, 0)),
        out_shape=jax.ShapeDtypeStruct(src_seqs.shape, jnp.float32),
    )(src_seqs, Wt, params)
